# baseline (speedup 1.0000x reference)
import jax
import jax.numpy as jnp
from jax.experimental import pallas as pl

_BN = 4096


def _body(x_ref, wt_ref, c_ref, o_ref):
    z = jax.lax.dot_general(
        x_ref[...], wt_ref[...],
        (((1,), (0,)), ((), ())),
        preferred_element_type=jnp.float32,
    )
    c1 = c_ref[0, 0]
    c2 = c_ref[0, 1]
    o_ref[...] = z * c1 + jnp.abs(z) * c2


def kernel(last, W, b, a):
    Bt, N, D = last.shape
    O = W.shape[0]
    rows = Bt * N
    x = last.reshape(rows, D)
    ones = jnp.ones((rows, 1), dtype=last.dtype)
    x4 = jnp.concatenate([x, ones], axis=1)
    wt = jnp.concatenate([W.T, b[None, :]], axis=0)
    av = a[0]
    cc = jnp.stack([(1.0 + av) * 0.5, (1.0 - av) * 0.5]).reshape(1, 2)

    grid = (rows // _BN,)
    out = pl.pallas_call(
        _body,
        grid=grid,
        in_specs=[
            pl.BlockSpec((_BN, D + 1), lambda i: (i, 0)),
            pl.BlockSpec((D + 1, O), lambda i: (0, 0)),
            pl.BlockSpec((1, 2), lambda i: (0, 0)),
        ],
        out_specs=pl.BlockSpec((_BN, O), lambda i: (i, 0)),
        out_shape=jax.ShapeDtypeStruct((rows, O), last.dtype),
    )(x4, wt, cc)
    return out.reshape(Bt, N, O)


# speedup vs baseline: 1.1159x; 1.1159x over previous
import jax
import jax.numpy as jnp
from jax.experimental import pallas as pl

_BN = 4096


def _body(x_ref, wt_ref, b_ref, c_ref, o_ref):
    z = jax.lax.dot_general(
        x_ref[...], wt_ref[...],
        (((1,), (0,)), ((), ())),
        preferred_element_type=jnp.float32,
    )
    z = z + b_ref[...]
    c1 = c_ref[0, 0]
    c2 = c_ref[0, 1]
    o_ref[...] = z * c1 + jnp.abs(z) * c2


def kernel(last, W, b, a):
    Bt, N, D = last.shape
    O = W.shape[0]
    rows = Bt * N
    x = last.reshape(rows, D)
    wt = W.T
    bb = b.reshape(1, O)
    av = a[0]
    cc = jnp.stack([(1.0 + av) * 0.5, (1.0 - av) * 0.5]).reshape(1, 2)

    grid = (rows // _BN,)
    out = pl.pallas_call(
        _body,
        grid=grid,
        in_specs=[
            pl.BlockSpec((_BN, D), lambda i: (i, 0)),
            pl.BlockSpec((D, O), lambda i: (0, 0)),
            pl.BlockSpec((1, O), lambda i: (0, 0)),
            pl.BlockSpec((1, 2), lambda i: (0, 0)),
        ],
        out_specs=pl.BlockSpec((_BN, O), lambda i: (i, 0)),
        out_shape=jax.ShapeDtypeStruct((rows, O), last.dtype),
    )(x, wt, bb, cc)
    return out.reshape(Bt, N, O)


# speedup vs baseline: 5.8220x; 5.2173x over previous
import jax
import jax.numpy as jnp
from jax.experimental import pallas as pl

_BN = 4096


def _body(x_ref, w_ref, b_ref, a_ref, o_ref):
    z = jax.lax.dot_general(
        x_ref[0], w_ref[...],
        (((1,), (1,)), ((), ())),
        preferred_element_type=jnp.float32,
    )
    z = z + b_ref[...]
    av = a_ref[0, 0]
    c1 = (1.0 + av) * 0.5
    c2 = (1.0 - av) * 0.5
    o_ref[0] = z * c1 + jnp.abs(z) * c2


def kernel(last, W, b, a):
    Bt, N, D = last.shape
    O = W.shape[0]
    bb = b.reshape(1, O)
    aa = a.reshape(1, 1)

    grid = (Bt, N // _BN)
    out = pl.pallas_call(
        _body,
        grid=grid,
        in_specs=[
            pl.BlockSpec((1, _BN, D), lambda i, j: (i, j, 0)),
            pl.BlockSpec((O, D), lambda i, j: (0, 0)),
            pl.BlockSpec((1, O), lambda i, j: (0, 0)),
            pl.BlockSpec((1, 1), lambda i, j: (0, 0)),
        ],
        out_specs=pl.BlockSpec((1, _BN, O), lambda i, j: (i, j, 0)),
        out_shape=jax.ShapeDtypeStruct((Bt, N, O), last.dtype),
    )(last, W, bb, aa)
    return out


# speedup vs baseline: 10.4510x; 1.7951x over previous
import jax
import jax.numpy as jnp
from jax.experimental import pallas as pl

_BN = 4096


def _body(x_ref, w_ref, b_ref, a_ref, o_ref):
    z = jax.lax.dot_general(
        x_ref[0], w_ref[...],
        (((0,), (1,)), ((), ())),
        preferred_element_type=jnp.float32,
    )
    z = z + b_ref[...]
    av = a_ref[0, 0]
    c1 = (1.0 + av) * 0.5
    c2 = (1.0 - av) * 0.5
    o_ref[0] = z * c1 + jnp.abs(z) * c2


def kernel(last, W, b, a):
    Bt, N, D = last.shape
    O = W.shape[0]
    xt = last.transpose(0, 2, 1)
    bb = b.reshape(1, O)
    aa = a.reshape(1, 1)

    grid = (Bt, N // _BN)
    out = pl.pallas_call(
        _body,
        grid=grid,
        in_specs=[
            pl.BlockSpec((1, D, _BN), lambda i, j: (i, 0, j)),
            pl.BlockSpec((O, D), lambda i, j: (0, 0)),
            pl.BlockSpec((1, O), lambda i, j: (0, 0)),
            pl.BlockSpec((1, 1), lambda i, j: (0, 0)),
        ],
        out_specs=pl.BlockSpec((1, _BN, O), lambda i, j: (i, j, 0)),
        out_shape=jax.ShapeDtypeStruct((Bt, N, O), last.dtype),
    )(xt, W, bb, aa)
    return out


# speedup vs baseline: 13.0573x; 1.2494x over previous
import jax
import jax.numpy as jnp
from jax.experimental import pallas as pl

_BN = 8192


def _body(x_ref, w_ref, b_ref, c_ref, o_ref):
    z = jax.lax.dot_general(
        x_ref[0], w_ref[...],
        (((0,), (1,)), ((), ())),
        preferred_element_type=jnp.float32,
    )
    z = z + b_ref[...]
    o_ref[0] = z + c_ref[0, 0] * jnp.abs(z)


def kernel(last, W, b, a):
    Bt, N, D = last.shape
    O = W.shape[0]
    xt = last.transpose(0, 2, 1)
    av = a[0]
    c1 = (1.0 + av) * 0.5
    ws = W * c1
    bs = (b * c1).reshape(1, O)
    cc = ((1.0 - av) / (1.0 + av)).reshape(1, 1)

    grid = (Bt, N // _BN)
    out = pl.pallas_call(
        _body,
        grid=grid,
        in_specs=[
            pl.BlockSpec((1, D, _BN), lambda i, j: (i, 0, j)),
            pl.BlockSpec((O, D), lambda i, j: (0, 0)),
            pl.BlockSpec((1, O), lambda i, j: (0, 0)),
            pl.BlockSpec((1, 1), lambda i, j: (0, 0)),
        ],
        out_specs=pl.BlockSpec((1, _BN, O), lambda i, j: (i, j, 0)),
        out_shape=jax.ShapeDtypeStruct((Bt, N, O), last.dtype),
    )(xt, ws, bs, cc)
    return out
